# baseline (speedup 1.0000x reference)
import functools

import jax
import jax.numpy as jnp
import numpy as np
from jax.experimental import pallas as pl
from jax.experimental.pallas import tpu as pltpu

E = 3_200_000
D = 16
H = 32
K = 8
G = 8
E8 = E // G
ROWS = 2_000


def _gater_kernel(x_ref, w1_ref, b1_ref, w2_ref, b2_ref, wp_ref,
                  gsum_ref, sel_ref, alpha_ref, fused_ref):
    x = x_ref[...]
    h = jnp.dot(x, w1_ref[...], preferred_element_type=jnp.float32)
    h = jnp.maximum(h + b1_ref[...], 0.0)
    logits = jnp.dot(h, w2_ref[...], preferred_element_type=jnp.float32)
    logits = logits + b2_ref[...]
    m = jnp.max(logits, axis=-1, keepdims=True)
    ex = jnp.exp(logits - m)
    denom = jnp.dot(ex, gsum_ref[...],
                    preferred_element_type=jnp.float32)
    alpha = ex / denom
    scores = jnp.dot(x, wp_ref[...],
                     preferred_element_type=jnp.float32)
    alpha_ref[...] = alpha
    fused_ref[...] = jnp.dot(alpha * scores, sel_ref[...],
                             preferred_element_type=jnp.float32)


@jax.jit
def kernel(edge_features, W1, b1, W2, b2, Wp):
    f32 = jnp.float32
    x = edge_features.reshape(E8, G * D)

    eye = jnp.eye(G, dtype=f32)
    w1b = jnp.kron(eye, W1)
    w2b = jnp.kron(eye, W2)
    wpb = jnp.kron(eye, Wp)
    b1b = jnp.tile(b1, G).reshape(1, G * H)
    b2b = jnp.tile(b2, G).reshape(1, G * K)
    gsum = jnp.kron(eye, jnp.ones((K, K), dtype=f32))
    sel = jnp.kron(eye, jnp.ones((K, 1), dtype=f32))

    def const(shape):
        return pl.BlockSpec(shape, lambda i: (0, 0))

    alpha_r, fused_r = pl.pallas_call(
        _gater_kernel,
        grid=(E8 // ROWS,),
        in_specs=[
            pl.BlockSpec((ROWS, G * D), lambda i: (i, 0)),
            const((G * D, G * H)), const((1, G * H)),
            const((G * H, G * K)), const((1, G * K)),
            const((G * D, G * K)),
            const((G * K, G * K)), const((G * K, K)),
        ],
        out_specs=[
            pl.BlockSpec((ROWS, G * K), lambda i: (i, 0)),
            pl.BlockSpec((ROWS, K), lambda i: (i, 0)),
        ],
        out_shape=[
            jax.ShapeDtypeStruct((E8, G * K), f32),
            jax.ShapeDtypeStruct((E8, K), f32),
        ],
        compiler_params=pltpu.CompilerParams(
            dimension_semantics=("arbitrary",)),
    )(x, w1b, b1b, w2b, b2b, wpb, gsum, sel)

    return fused_r.reshape(E), alpha_r.reshape(E, K)
